# baseline (speedup 1.0000x reference)
import functools

import jax
import jax.numpy as jnp
from jax import lax
from jax.experimental import pallas as pl
from jax.experimental.pallas import tpu as pltpu
from jax.experimental.pallas import tpu_sc as plsc

NC = 2
NS = 16
C = 128
NB = 2


def _sc_aggregate(table, srcw, dstw, n_pad):
    D = table.shape[1]
    K = srcw.shape[1] - NB
    rpt = n_pad // NS
    zeros = jnp.zeros((n_pad, D), jnp.float32)
    mesh = plsc.VectorSubcoreMesh(core_axis_name="c", subcore_axis_name="s")

    @functools.partial(
        pl.kernel,
        out_type=jax.ShapeDtypeStruct((NC, n_pad, D), jnp.float32),
        mesh=mesh,
        scratch_types=[
            pltpu.VMEM((K + NB, C), jnp.int32),
            pltpu.VMEM((K + NB, C), jnp.int32),
            pltpu.VMEM((NB, C, D), jnp.float32),
            pltpu.VMEM_SHARED((n_pad, D), jnp.float32),
        ] + [pltpu.SemaphoreType.DMA] * (2 * NB),
        compiler_params=pltpu.CompilerParams(use_tc_tiling_on_sc=False),
    )
    def agg(table_hbm, srcw_hbm, dstw_hbm, zeros_hbm, out_hbm,
            srcv, dstv, rows, acc, *sems):
        gsem, ssem = sems[:NB], sems[NB:]
        cid = lax.axis_index("c")
        sid = lax.axis_index("s")
        w = cid * NS + sid
        pltpu.sync_copy(zeros_hbm.at[pl.ds(sid * rpt, rpt)],
                        acc.at[pl.ds(sid * rpt, rpt)])
        pltpu.sync_copy(srcw_hbm.at[w], srcv)
        pltpu.sync_copy(dstw_hbm.at[w], dstv)
        plsc.subcore_barrier()

        for b in range(NB):
            pltpu.async_copy(table_hbm.at[srcv.at[b]], rows.at[b], gsem[b])

        def round_(i, carry):
            j0 = i * NB
            for b in range(NB):
                j = j0 + b
                pltpu.make_async_copy(
                    table_hbm.at[srcv.at[j]], rows.at[b], gsem[b]).wait()
                pltpu.async_copy(
                    rows.at[b], acc.at[dstv.at[j]], ssem[b], add=True)
            for b in range(NB):
                j = j0 + b
                pltpu.make_async_copy(
                    rows.at[b], acc.at[dstv.at[j]], ssem[b]).wait()
                pltpu.async_copy(
                    table_hbm.at[srcv.at[j + NB]], rows.at[b], gsem[b])
            return carry

        lax.fori_loop(0, K // NB, round_, 0)
        for b in range(NB):
            pltpu.make_async_copy(
                table_hbm.at[srcv.at[K + b]], rows.at[b], gsem[b]).wait()
        plsc.subcore_barrier()
        pltpu.sync_copy(acc.at[pl.ds(sid * rpt, rpt)],
                        out_hbm.at[cid, pl.ds(sid * rpt, rpt)])

    return agg(table, srcw, dstw, zeros)


def _sc_degree(dstw, n_pad):
    K = dstw.shape[1] - NB
    DD = 8
    rpt = n_pad // NS
    zeros = jnp.zeros((n_pad, DD), jnp.float32)
    ones = jnp.ones((C, DD), jnp.float32)
    mesh = plsc.VectorSubcoreMesh(core_axis_name="c", subcore_axis_name="s")

    @functools.partial(
        pl.kernel,
        out_type=jax.ShapeDtypeStruct((NC, n_pad, DD), jnp.float32),
        mesh=mesh,
        scratch_types=[
            pltpu.VMEM((K + NB, C), jnp.int32),
            pltpu.VMEM((C, DD), jnp.float32),
            pltpu.VMEM_SHARED((n_pad, DD), jnp.float32),
        ] + [pltpu.SemaphoreType.DMA] * NB,
        compiler_params=pltpu.CompilerParams(use_tc_tiling_on_sc=False),
    )
    def deg(dstw_hbm, zeros_hbm, ones_hbm, out_hbm, dstv, onesv, acc, *ssem):
        cid = lax.axis_index("c")
        sid = lax.axis_index("s")
        w = cid * NS + sid
        pltpu.sync_copy(zeros_hbm.at[pl.ds(sid * rpt, rpt)],
                        acc.at[pl.ds(sid * rpt, rpt)])
        pltpu.sync_copy(dstw_hbm.at[w], dstv)
        pltpu.sync_copy(ones_hbm, onesv)
        plsc.subcore_barrier()

        for b in range(NB):
            pltpu.async_copy(onesv, acc.at[dstv.at[b]], ssem[b], add=True)

        def round_(i, carry):
            j0 = i * NB
            for b in range(NB):
                j = j0 + b
                pltpu.make_async_copy(
                    onesv, acc.at[dstv.at[j]], ssem[b]).wait()
                pltpu.async_copy(
                    onesv, acc.at[dstv.at[j + NB]], ssem[b], add=True)
            return carry

        lax.fori_loop(0, K // NB, round_, 0)
        for b in range(NB):
            pltpu.make_async_copy(
                onesv, acc.at[dstv.at[K + b]], ssem[b]).wait()
        plsc.subcore_barrier()
        pltpu.sync_copy(acc.at[pl.ds(sid * rpt, rpt)],
                        out_hbm.at[cid, pl.ds(sid * rpt, rpt)])

    return deg(dstw, zeros, ones)


def _dinv_block(degp):
    deg = degp[0, :, 0:1] + degp[1, :, 0:1] + 1.0
    return lax.rsqrt(deg)


def _row_block(n):
    for r in (2000, 1600, 1250, 1000, 800, 640, 625, 500, 400, 250, 200, 125, 100):
        if n % r == 0:
            return r
    return n


def _tc_layer1(x, W1, degp, n_pad):
    N, D_IN = x.shape
    D_HID = W1.shape[1]
    R = _row_block(N)

    def body(x_ref, w1_ref, degp_ref, hs_ref):
        dinv = _dinv_block(degp_ref[...])
        h = jnp.dot(x_ref[...], w1_ref[...], preferred_element_type=jnp.float32)
        hs_ref[...] = h * dinv

    return pl.pallas_call(
        body,
        grid=(N // R,),
        in_specs=[
            pl.BlockSpec((R, D_IN), lambda j: (j, 0)),
            pl.BlockSpec((D_IN, D_HID), lambda j: (0, 0)),
            pl.BlockSpec((NC, R, 8), lambda j: (0, j, 0)),
        ],
        out_specs=pl.BlockSpec((R, D_HID), lambda j: (j, 0)),
        out_shape=jax.ShapeDtypeStruct((N, D_HID), jnp.float32),
    )(x, W1, degp)


def _tc_layer2(hs, aggp, degp, b1, W2, n_pad):
    N, D_HID = hs.shape
    D_OUT = W2.shape[1]
    R = _row_block(N)

    def body(hs_ref, aggp_ref, degp_ref, b1_ref, w2_ref, ts_ref):
        dinv = _dinv_block(degp_ref[...])
        s = aggp_ref[0] + aggp_ref[1] + hs_ref[...]
        h1 = jnp.maximum(s * dinv + b1_ref[...], 0.0)
        t = jnp.dot(h1, w2_ref[...], preferred_element_type=jnp.float32)
        ts_ref[...] = t * dinv

    return pl.pallas_call(
        body,
        grid=(N // R,),
        in_specs=[
            pl.BlockSpec((R, D_HID), lambda j: (j, 0)),
            pl.BlockSpec((NC, R, D_HID), lambda j: (0, j, 0)),
            pl.BlockSpec((NC, R, 8), lambda j: (0, j, 0)),
            pl.BlockSpec((1, D_HID), lambda j: (0, 0)),
            pl.BlockSpec((D_HID, D_OUT), lambda j: (0, 0)),
        ],
        out_specs=pl.BlockSpec((R, D_OUT), lambda j: (j, 0)),
        out_shape=jax.ShapeDtypeStruct((N, D_OUT), jnp.float32),
    )(hs, aggp, degp, b1.reshape(1, D_HID), W2)


def _tc_head(ts, aggp, degp, b2, Wfc, bfc, n_pad):
    N, D_OUT = ts.shape
    R = _row_block(N)
    G = N // R

    def body(ts_ref, aggp_ref, degp_ref, b2_ref, wfc_ref, bfc_ref, out_ref, acc_ref):
        j = pl.program_id(0)
        dinv = _dinv_block(degp_ref[...])
        s = aggp_ref[0] + aggp_ref[1] + ts_ref[...]
        h2 = jnp.maximum(s * dinv + b2_ref[...], 0.0)
        csum = jnp.sum(h2, axis=0, keepdims=True)

        @pl.when(j == 0)
        def _():
            acc_ref[...] = csum

        @pl.when(j > 0)
        def _():
            acc_ref[...] += csum

        @pl.when(j == G - 1)
        def _():
            g = acc_ref[...] * (1.0 / N)
            z = jnp.dot(g, wfc_ref[...], preferred_element_type=jnp.float32)
            z = z + bfc_ref[...]
            out_ref[...] = 1.0 / (1.0 + jnp.exp(-z))

    return pl.pallas_call(
        body,
        grid=(G,),
        in_specs=[
            pl.BlockSpec((R, D_OUT), lambda j: (j, 0)),
            pl.BlockSpec((NC, R, D_OUT), lambda j: (0, j, 0)),
            pl.BlockSpec((NC, R, 8), lambda j: (0, j, 0)),
            pl.BlockSpec((1, D_OUT), lambda j: (0, 0)),
            pl.BlockSpec((D_OUT, 1), lambda j: (0, 0)),
            pl.BlockSpec((1, 1), lambda j: (0, 0)),
        ],
        out_specs=pl.BlockSpec((1, 1), lambda j: (0, 0)),
        out_shape=jax.ShapeDtypeStruct((1, 1), jnp.float32),
        scratch_shapes=[pltpu.VMEM((1, D_OUT), jnp.float32)],
    )(ts, aggp, degp, b2.reshape(1, D_OUT), Wfc, bfc.reshape(1, 1))


def kernel(x, edge_index, W1, b1, W2, b2, Wfc, bfc):
    N = x.shape[0]
    E = edge_index.shape[1]
    NW = NC * NS
    K = -(-E // (NW * C))
    if K % NB:
        K += NB - K % NB
    e_pad = NW * K * C
    n_pad = -(-(N + 1) // 128) * 128

    src = edge_index[0]
    dst = edge_index[1]
    src_p = jnp.concatenate(
        [src, jnp.zeros((e_pad - E,), jnp.int32)]).reshape(NW, K, C)
    dst_p = jnp.concatenate(
        [dst, jnp.full((e_pad - E,), N, jnp.int32)]).reshape(NW, K, C)
    src_p = jnp.concatenate(
        [src_p, jnp.zeros((NW, NB, C), jnp.int32)], axis=1)
    dst_p = jnp.concatenate(
        [dst_p, jnp.full((NW, NB, C), N, jnp.int32)], axis=1)

    degp = _sc_degree(dst_p, n_pad)
    hs = _tc_layer1(x, W1, degp, n_pad)
    agg1 = _sc_aggregate(hs, src_p, dst_p, n_pad)
    ts = _tc_layer2(hs, agg1, degp, b1, W2, n_pad)
    agg2 = _sc_aggregate(ts, src_p, dst_p, n_pad)
    out = _tc_head(ts, agg2, degp, b2, Wfc, bfc, n_pad)
    return out.reshape(1)


# speedup vs baseline: 1.5009x; 1.5009x over previous
import functools

import jax
import jax.numpy as jnp
from jax import lax
from jax.experimental import pallas as pl
from jax.experimental.pallas import tpu as pltpu
from jax.experimental.pallas import tpu_sc as plsc

NC = 2
NS = 16
C = 512
NB = 2


def _sc_aggregate(table, srcw, dstw, n_pad):
    D = table.shape[1]
    K = srcw.shape[1] - NB
    rpt = n_pad // NS
    zeros = jnp.zeros((n_pad, D), jnp.float32)
    mesh = plsc.VectorSubcoreMesh(core_axis_name="c", subcore_axis_name="s")

    @functools.partial(
        pl.kernel,
        out_type=jax.ShapeDtypeStruct((NC, n_pad, D), jnp.float32),
        mesh=mesh,
        scratch_types=[
            pltpu.VMEM((K + NB, C), jnp.int32),
            pltpu.VMEM((K + NB, C), jnp.int32),
            pltpu.VMEM((1, C, D), jnp.float32),
            pltpu.VMEM_SHARED((n_pad, D), jnp.float32),
        ] + [pltpu.SemaphoreType.DMA],
        compiler_params=pltpu.CompilerParams(use_tc_tiling_on_sc=False),
    )
    def agg(table_hbm, srcw_hbm, dstw_hbm, zeros_hbm, out_hbm,
            srcv, dstv, rows, acc, *sems):
        gsem = sems
        cid = lax.axis_index("c")
        sid = lax.axis_index("s")
        w = cid * NS + sid
        pltpu.sync_copy(zeros_hbm.at[pl.ds(sid * rpt, rpt)],
                        acc.at[pl.ds(sid * rpt, rpt)])
        pltpu.sync_copy(srcw_hbm.at[w], srcv)
        pltpu.sync_copy(dstw_hbm.at[w], dstv)
        plsc.subcore_barrier()

        def chunk(j, carry):
            pltpu.async_copy(
                table_hbm.at[srcv.at[j]], rows.at[0], gsem[0]).wait()
            pltpu.sync_copy(rows.at[0], acc.at[dstv.at[j]], add=True)
            return carry

        lax.fori_loop(0, K, chunk, 0)
        plsc.subcore_barrier()
        pltpu.sync_copy(acc.at[pl.ds(sid * rpt, rpt)],
                        out_hbm.at[cid, pl.ds(sid * rpt, rpt)])

    return agg(table, srcw, dstw, zeros)


def _sc_degree(dstw, n_pad):
    K = dstw.shape[1] - NB
    DD = 8
    rpt = n_pad // NS
    zeros = jnp.zeros((n_pad, DD), jnp.float32)
    ones = jnp.ones((C, DD), jnp.float32)
    mesh = plsc.VectorSubcoreMesh(core_axis_name="c", subcore_axis_name="s")

    @functools.partial(
        pl.kernel,
        out_type=jax.ShapeDtypeStruct((NC, n_pad, DD), jnp.float32),
        mesh=mesh,
        scratch_types=[
            pltpu.VMEM((K + NB, C), jnp.int32),
            pltpu.VMEM((C, DD), jnp.float32),
            pltpu.VMEM_SHARED((n_pad, DD), jnp.float32),
        ],
        compiler_params=pltpu.CompilerParams(use_tc_tiling_on_sc=False),
    )
    def deg(dstw_hbm, zeros_hbm, ones_hbm, out_hbm, dstv, onesv, acc):
        cid = lax.axis_index("c")
        sid = lax.axis_index("s")
        w = cid * NS + sid
        pltpu.sync_copy(zeros_hbm.at[pl.ds(sid * rpt, rpt)],
                        acc.at[pl.ds(sid * rpt, rpt)])
        pltpu.sync_copy(dstw_hbm.at[w], dstv)
        pltpu.sync_copy(ones_hbm, onesv)
        plsc.subcore_barrier()

        def chunk(j, carry):
            pltpu.sync_copy(onesv, acc.at[dstv.at[j]], add=True)
            return carry

        lax.fori_loop(0, K, chunk, 0)
        plsc.subcore_barrier()
        pltpu.sync_copy(acc.at[pl.ds(sid * rpt, rpt)],
                        out_hbm.at[cid, pl.ds(sid * rpt, rpt)])

    return deg(dstw, zeros, ones)


def _dinv_block(degp):
    deg = degp[0, :, 0:1] + degp[1, :, 0:1] + 1.0
    return lax.rsqrt(deg)


def _row_block(n):
    for r in (2000, 1600, 1250, 1000, 800, 640, 625, 500, 400, 250, 200, 125, 100):
        if n % r == 0:
            return r
    return n


def _tc_layer1(x, W1, degp, n_pad):
    N, D_IN = x.shape
    D_HID = W1.shape[1]
    R = _row_block(N)

    def body(x_ref, w1_ref, degp_ref, hs_ref):
        dinv = _dinv_block(degp_ref[...])
        h = jnp.dot(x_ref[...], w1_ref[...], preferred_element_type=jnp.float32)
        hs_ref[...] = h * dinv

    return pl.pallas_call(
        body,
        grid=(N // R,),
        in_specs=[
            pl.BlockSpec((R, D_IN), lambda j: (j, 0)),
            pl.BlockSpec((D_IN, D_HID), lambda j: (0, 0)),
            pl.BlockSpec((NC, R, 8), lambda j: (0, j, 0)),
        ],
        out_specs=pl.BlockSpec((R, D_HID), lambda j: (j, 0)),
        out_shape=jax.ShapeDtypeStruct((N, D_HID), jnp.float32),
    )(x, W1, degp)


def _tc_layer2(hs, aggp, degp, b1, W2, n_pad):
    N, D_HID = hs.shape
    D_OUT = W2.shape[1]
    R = _row_block(N)

    def body(hs_ref, aggp_ref, degp_ref, b1_ref, w2_ref, ts_ref):
        dinv = _dinv_block(degp_ref[...])
        s = aggp_ref[0] + aggp_ref[1] + hs_ref[...]
        h1 = jnp.maximum(s * dinv + b1_ref[...], 0.0)
        t = jnp.dot(h1, w2_ref[...], preferred_element_type=jnp.float32)
        ts_ref[...] = t * dinv

    return pl.pallas_call(
        body,
        grid=(N // R,),
        in_specs=[
            pl.BlockSpec((R, D_HID), lambda j: (j, 0)),
            pl.BlockSpec((NC, R, D_HID), lambda j: (0, j, 0)),
            pl.BlockSpec((NC, R, 8), lambda j: (0, j, 0)),
            pl.BlockSpec((1, D_HID), lambda j: (0, 0)),
            pl.BlockSpec((D_HID, D_OUT), lambda j: (0, 0)),
        ],
        out_specs=pl.BlockSpec((R, D_OUT), lambda j: (j, 0)),
        out_shape=jax.ShapeDtypeStruct((N, D_OUT), jnp.float32),
    )(hs, aggp, degp, b1.reshape(1, D_HID), W2)


def _tc_head(ts, aggp, degp, b2, Wfc, bfc, n_pad):
    N, D_OUT = ts.shape
    R = _row_block(N)
    G = N // R

    def body(ts_ref, aggp_ref, degp_ref, b2_ref, wfc_ref, bfc_ref, out_ref, acc_ref):
        j = pl.program_id(0)
        dinv = _dinv_block(degp_ref[...])
        s = aggp_ref[0] + aggp_ref[1] + ts_ref[...]
        h2 = jnp.maximum(s * dinv + b2_ref[...], 0.0)
        csum = jnp.sum(h2, axis=0, keepdims=True)

        @pl.when(j == 0)
        def _():
            acc_ref[...] = csum

        @pl.when(j > 0)
        def _():
            acc_ref[...] += csum

        @pl.when(j == G - 1)
        def _():
            g = acc_ref[...] * (1.0 / N)
            z = jnp.dot(g, wfc_ref[...], preferred_element_type=jnp.float32)
            z = z + bfc_ref[...]
            out_ref[...] = 1.0 / (1.0 + jnp.exp(-z))

    return pl.pallas_call(
        body,
        grid=(G,),
        in_specs=[
            pl.BlockSpec((R, D_OUT), lambda j: (j, 0)),
            pl.BlockSpec((NC, R, D_OUT), lambda j: (0, j, 0)),
            pl.BlockSpec((NC, R, 8), lambda j: (0, j, 0)),
            pl.BlockSpec((1, D_OUT), lambda j: (0, 0)),
            pl.BlockSpec((D_OUT, 1), lambda j: (0, 0)),
            pl.BlockSpec((1, 1), lambda j: (0, 0)),
        ],
        out_specs=pl.BlockSpec((1, 1), lambda j: (0, 0)),
        out_shape=jax.ShapeDtypeStruct((1, 1), jnp.float32),
        scratch_shapes=[pltpu.VMEM((1, D_OUT), jnp.float32)],
    )(ts, aggp, degp, b2.reshape(1, D_OUT), Wfc, bfc.reshape(1, 1))


def kernel(x, edge_index, W1, b1, W2, b2, Wfc, bfc):
    N = x.shape[0]
    E = edge_index.shape[1]
    NW = NC * NS
    K = -(-E // (NW * C))
    if K % NB:
        K += NB - K % NB
    e_pad = NW * K * C
    n_pad = -(-(N + 1) // 128) * 128

    src = edge_index[0]
    dst = edge_index[1]
    src_p = jnp.concatenate(
        [src, jnp.zeros((e_pad - E,), jnp.int32)]).reshape(NW, K, C)
    dst_p = jnp.concatenate(
        [dst, jnp.full((e_pad - E,), N, jnp.int32)]).reshape(NW, K, C)
    src_p = jnp.concatenate(
        [src_p, jnp.zeros((NW, NB, C), jnp.int32)], axis=1)
    dst_p = jnp.concatenate(
        [dst_p, jnp.full((NW, NB, C), N, jnp.int32)], axis=1)

    degp = _sc_degree(dst_p, n_pad)
    hs = _tc_layer1(x, W1, degp, n_pad)
    agg1 = _sc_aggregate(hs, src_p, dst_p, n_pad)
    ts = _tc_layer2(hs, agg1, degp, b1, W2, n_pad)
    agg2 = _sc_aggregate(ts, src_p, dst_p, n_pad)
    out = _tc_head(ts, agg2, degp, b2, Wfc, bfc, n_pad)
    return out.reshape(1)


# speedup vs baseline: 2.5603x; 1.7058x over previous
import functools

import jax
import jax.numpy as jnp
from jax import lax
from jax.experimental import pallas as pl
from jax.experimental.pallas import tpu as pltpu
from jax.experimental.pallas import tpu_sc as plsc

NC = 2
NS = 16
C_AGG = 256
C_DEG = 512


def _sc_aggregate(table, srcw, dstw, n_pad):
    C = C_AGG
    D = table.shape[1]
    NT = table.shape[0]
    tpt = NT // NS
    K = srcw.shape[1]
    rpt = n_pad // NS
    zeros = jnp.zeros((n_pad, D), jnp.float32)
    mesh = plsc.VectorSubcoreMesh(core_axis_name="c", subcore_axis_name="s")

    @functools.partial(
        pl.kernel,
        out_type=jax.ShapeDtypeStruct((NC, n_pad, D), jnp.float32),
        mesh=mesh,
        scratch_types=[
            pltpu.VMEM((K, C), jnp.int32),
            pltpu.VMEM((K, C), jnp.int32),
            pltpu.VMEM((1, C, D), jnp.float32),
            pltpu.VMEM_SHARED((NT, D), jnp.float32),
            pltpu.VMEM_SHARED((n_pad, D), jnp.float32),
        ] + [pltpu.SemaphoreType.DMA],
        compiler_params=pltpu.CompilerParams(use_tc_tiling_on_sc=False),
    )
    def agg(table_hbm, srcw_hbm, dstw_hbm, zeros_hbm, out_hbm,
            srcv, dstv, rows, tbl, acc, *sems):
        gsem = sems
        cid = lax.axis_index("c")
        sid = lax.axis_index("s")
        w = cid * NS + sid
        pltpu.sync_copy(table_hbm.at[pl.ds(sid * tpt, tpt)],
                        tbl.at[pl.ds(sid * tpt, tpt)])
        pltpu.sync_copy(zeros_hbm.at[pl.ds(sid * rpt, rpt)],
                        acc.at[pl.ds(sid * rpt, rpt)])
        pltpu.sync_copy(srcw_hbm.at[w], srcv)
        pltpu.sync_copy(dstw_hbm.at[w], dstv)
        plsc.subcore_barrier()

        def chunk(j, carry):
            pltpu.async_copy(
                tbl.at[srcv.at[j]], rows.at[0], gsem[0]).wait()
            pltpu.sync_copy(rows.at[0], acc.at[dstv.at[j]], add=True)
            return carry

        lax.fori_loop(0, K, chunk, 0)
        plsc.subcore_barrier()
        pltpu.sync_copy(acc.at[pl.ds(sid * rpt, rpt)],
                        out_hbm.at[cid, pl.ds(sid * rpt, rpt)])

    return agg(table, srcw, dstw, zeros)


def _sc_degree(dstw, n_pad):
    C = C_DEG
    K = dstw.shape[1]
    DD = 8
    rpt = n_pad // NS
    zeros = jnp.zeros((n_pad, DD), jnp.float32)
    ones = jnp.ones((C, DD), jnp.float32)
    mesh = plsc.VectorSubcoreMesh(core_axis_name="c", subcore_axis_name="s")

    @functools.partial(
        pl.kernel,
        out_type=jax.ShapeDtypeStruct((NC, n_pad, DD), jnp.float32),
        mesh=mesh,
        scratch_types=[
            pltpu.VMEM((K, C), jnp.int32),
            pltpu.VMEM((C, DD), jnp.float32),
            pltpu.VMEM_SHARED((n_pad, DD), jnp.float32),
        ],
        compiler_params=pltpu.CompilerParams(use_tc_tiling_on_sc=False),
    )
    def deg(dstw_hbm, zeros_hbm, ones_hbm, out_hbm, dstv, onesv, acc):
        cid = lax.axis_index("c")
        sid = lax.axis_index("s")
        w = cid * NS + sid
        pltpu.sync_copy(zeros_hbm.at[pl.ds(sid * rpt, rpt)],
                        acc.at[pl.ds(sid * rpt, rpt)])
        pltpu.sync_copy(dstw_hbm.at[w], dstv)
        pltpu.sync_copy(ones_hbm, onesv)
        plsc.subcore_barrier()

        def chunk(j, carry):
            pltpu.sync_copy(onesv, acc.at[dstv.at[j]], add=True)
            return carry

        lax.fori_loop(0, K, chunk, 0)
        plsc.subcore_barrier()
        pltpu.sync_copy(acc.at[pl.ds(sid * rpt, rpt)],
                        out_hbm.at[cid, pl.ds(sid * rpt, rpt)])

    return deg(dstw, zeros, ones)


def _dinv_block(degp):
    deg = degp[0, :, 0:1] + degp[1, :, 0:1] + 1.0
    return lax.rsqrt(deg)


def _row_block(n):
    for r in (2000, 1600, 1250, 1000, 800, 640, 625, 500, 400, 250, 200, 125, 100):
        if n % r == 0:
            return r
    return n


def _tc_layer1(x, W1, degp, n_pad):
    N, D_IN = x.shape
    D_HID = W1.shape[1]
    R = _row_block(N)

    def body(x_ref, w1_ref, degp_ref, hs_ref):
        dinv = _dinv_block(degp_ref[...])
        h = jnp.dot(x_ref[...], w1_ref[...], preferred_element_type=jnp.float32)
        hs_ref[...] = h * dinv

    return pl.pallas_call(
        body,
        grid=(N // R,),
        in_specs=[
            pl.BlockSpec((R, D_IN), lambda j: (j, 0)),
            pl.BlockSpec((D_IN, D_HID), lambda j: (0, 0)),
            pl.BlockSpec((NC, R, 8), lambda j: (0, j, 0)),
        ],
        out_specs=pl.BlockSpec((R, D_HID), lambda j: (j, 0)),
        out_shape=jax.ShapeDtypeStruct((N, D_HID), jnp.float32),
    )(x, W1, degp)


def _tc_layer2(hs, aggp, degp, b1, W2, n_pad):
    N, D_HID = hs.shape
    D_OUT = W2.shape[1]
    R = _row_block(N)

    def body(hs_ref, aggp_ref, degp_ref, b1_ref, w2_ref, ts_ref):
        dinv = _dinv_block(degp_ref[...])
        s = aggp_ref[0] + aggp_ref[1] + hs_ref[...]
        h1 = jnp.maximum(s * dinv + b1_ref[...], 0.0)
        t = jnp.dot(h1, w2_ref[...], preferred_element_type=jnp.float32)
        ts_ref[...] = t * dinv

    return pl.pallas_call(
        body,
        grid=(N // R,),
        in_specs=[
            pl.BlockSpec((R, D_HID), lambda j: (j, 0)),
            pl.BlockSpec((NC, R, D_HID), lambda j: (0, j, 0)),
            pl.BlockSpec((NC, R, 8), lambda j: (0, j, 0)),
            pl.BlockSpec((1, D_HID), lambda j: (0, 0)),
            pl.BlockSpec((D_HID, D_OUT), lambda j: (0, 0)),
        ],
        out_specs=pl.BlockSpec((R, D_OUT), lambda j: (j, 0)),
        out_shape=jax.ShapeDtypeStruct((N, D_OUT), jnp.float32),
    )(hs, aggp, degp, b1.reshape(1, D_HID), W2)


def _tc_head(ts, aggp, degp, b2, Wfc, bfc, n_pad):
    N, D_OUT = ts.shape
    R = _row_block(N)
    G = N // R

    def body(ts_ref, aggp_ref, degp_ref, b2_ref, wfc_ref, bfc_ref, out_ref, acc_ref):
        j = pl.program_id(0)
        dinv = _dinv_block(degp_ref[...])
        s = aggp_ref[0] + aggp_ref[1] + ts_ref[...]
        h2 = jnp.maximum(s * dinv + b2_ref[...], 0.0)
        csum = jnp.sum(h2, axis=0, keepdims=True)

        @pl.when(j == 0)
        def _():
            acc_ref[...] = csum

        @pl.when(j > 0)
        def _():
            acc_ref[...] += csum

        @pl.when(j == G - 1)
        def _():
            g = acc_ref[...] * (1.0 / N)
            z = jnp.dot(g, wfc_ref[...], preferred_element_type=jnp.float32)
            z = z + bfc_ref[...]
            out_ref[...] = 1.0 / (1.0 + jnp.exp(-z))

    return pl.pallas_call(
        body,
        grid=(G,),
        in_specs=[
            pl.BlockSpec((R, D_OUT), lambda j: (j, 0)),
            pl.BlockSpec((NC, R, D_OUT), lambda j: (0, j, 0)),
            pl.BlockSpec((NC, R, 8), lambda j: (0, j, 0)),
            pl.BlockSpec((1, D_OUT), lambda j: (0, 0)),
            pl.BlockSpec((D_OUT, 1), lambda j: (0, 0)),
            pl.BlockSpec((1, 1), lambda j: (0, 0)),
        ],
        out_specs=pl.BlockSpec((1, 1), lambda j: (0, 0)),
        out_shape=jax.ShapeDtypeStruct((1, 1), jnp.float32),
        scratch_shapes=[pltpu.VMEM((1, D_OUT), jnp.float32)],
    )(ts, aggp, degp, b2.reshape(1, D_OUT), Wfc, bfc.reshape(1, 1))


def kernel(x, edge_index, W1, b1, W2, b2, Wfc, bfc):
    N = x.shape[0]
    E = edge_index.shape[1]
    NW = NC * NS
    L = -(-E // (NW * C_DEG)) * C_DEG
    e_pad = NW * L
    n_pad = -(-(N + 1) // 128) * 128

    src = edge_index[0]
    dst = edge_index[1]
    src_f = jnp.concatenate(
        [src, jnp.zeros((e_pad - E,), jnp.int32)]).reshape(NW, L)
    dst_f = jnp.concatenate(
        [dst, jnp.full((e_pad - E,), N, jnp.int32)]).reshape(NW, L)
    src_p = src_f.reshape(NW, L // C_AGG, C_AGG)
    dst_p = dst_f.reshape(NW, L // C_AGG, C_AGG)
    dst_d = dst_f.reshape(NW, L // C_DEG, C_DEG)

    def pad_rows(a):
        nt = -(-a.shape[0] // NS) * NS
        if nt == a.shape[0]:
            return a
        return jnp.concatenate(
            [a, jnp.zeros((nt - a.shape[0], a.shape[1]), a.dtype)])

    degp = _sc_degree(dst_d, n_pad)
    hs = _tc_layer1(x, W1, degp, n_pad)
    agg1 = _sc_aggregate(pad_rows(hs), src_p, dst_p, n_pad)
    ts = _tc_layer2(hs, agg1, degp, b1, W2, n_pad)
    agg2 = _sc_aggregate(pad_rows(ts), src_p, dst_p, n_pad)
    out = _tc_head(ts, agg2, degp, b2, Wfc, bfc, n_pad)
    return out.reshape(1)


# speedup vs baseline: 3.6095x; 1.4098x over previous
import functools

import jax
import jax.numpy as jnp
from jax import lax
from jax.experimental import pallas as pl
from jax.experimental.pallas import tpu as pltpu
from jax.experimental.pallas import tpu_sc as plsc

NC = 2
NS = 16
C_AGG = 256
C_DEG = 512


def _sc_aggregate(table, srcw, dstw, n_pad):
    C = C_AGG
    D = table.shape[1]
    NT = table.shape[0]
    tpt = NT // NS
    K = srcw.shape[1]
    rpt = n_pad // NS
    dt = table.dtype
    zeros = jnp.zeros((n_pad, D), dt)
    mesh = plsc.VectorSubcoreMesh(core_axis_name="c", subcore_axis_name="s")

    @functools.partial(
        pl.kernel,
        out_type=jax.ShapeDtypeStruct((NC, n_pad, D), dt),
        mesh=mesh,
        scratch_types=[
            pltpu.VMEM((K, C), jnp.int32),
            pltpu.VMEM((K, C), jnp.int32),
            pltpu.VMEM((1, C, D), dt),
            pltpu.VMEM_SHARED((NT, D), dt),
            pltpu.VMEM_SHARED((n_pad, D), dt),
        ] + [pltpu.SemaphoreType.DMA],
        compiler_params=pltpu.CompilerParams(use_tc_tiling_on_sc=False),
    )
    def agg(table_hbm, srcw_hbm, dstw_hbm, zeros_hbm, out_hbm,
            srcv, dstv, rows, tbl, acc, *sems):
        gsem = sems
        cid = lax.axis_index("c")
        sid = lax.axis_index("s")
        w = cid * NS + sid
        pltpu.sync_copy(table_hbm.at[pl.ds(sid * tpt, tpt)],
                        tbl.at[pl.ds(sid * tpt, tpt)])
        pltpu.sync_copy(zeros_hbm.at[pl.ds(sid * rpt, rpt)],
                        acc.at[pl.ds(sid * rpt, rpt)])
        pltpu.sync_copy(srcw_hbm.at[w], srcv)
        pltpu.sync_copy(dstw_hbm.at[w], dstv)
        plsc.subcore_barrier()

        def chunk(j, carry):
            pltpu.async_copy(
                tbl.at[srcv.at[j]], rows.at[0], gsem[0]).wait()
            pltpu.sync_copy(rows.at[0], acc.at[dstv.at[j]], add=True)
            return carry

        lax.fori_loop(0, K, chunk, 0)
        plsc.subcore_barrier()
        pltpu.sync_copy(acc.at[pl.ds(sid * rpt, rpt)],
                        out_hbm.at[cid, pl.ds(sid * rpt, rpt)])

    return agg(table, srcw, dstw, zeros)


def _sc_degree(dstw, n_pad):
    C = C_DEG
    K = dstw.shape[1]
    DD = 8
    rpt = n_pad // NS
    zeros = jnp.zeros((n_pad, DD), jnp.float32)
    ones = jnp.ones((C, DD), jnp.float32)
    mesh = plsc.VectorSubcoreMesh(core_axis_name="c", subcore_axis_name="s")

    @functools.partial(
        pl.kernel,
        out_type=jax.ShapeDtypeStruct((NC, n_pad, DD), jnp.float32),
        mesh=mesh,
        scratch_types=[
            pltpu.VMEM((K, C), jnp.int32),
            pltpu.VMEM((C, DD), jnp.float32),
            pltpu.VMEM_SHARED((n_pad, DD), jnp.float32),
        ],
        compiler_params=pltpu.CompilerParams(use_tc_tiling_on_sc=False),
    )
    def deg(dstw_hbm, zeros_hbm, ones_hbm, out_hbm, dstv, onesv, acc):
        cid = lax.axis_index("c")
        sid = lax.axis_index("s")
        w = cid * NS + sid
        pltpu.sync_copy(zeros_hbm.at[pl.ds(sid * rpt, rpt)],
                        acc.at[pl.ds(sid * rpt, rpt)])
        pltpu.sync_copy(dstw_hbm.at[w], dstv)
        pltpu.sync_copy(ones_hbm, onesv)
        plsc.subcore_barrier()

        def chunk(j, carry):
            pltpu.sync_copy(onesv, acc.at[dstv.at[j]], add=True)
            return carry

        lax.fori_loop(0, K, chunk, 0)
        plsc.subcore_barrier()
        pltpu.sync_copy(acc.at[pl.ds(sid * rpt, rpt)],
                        out_hbm.at[cid, pl.ds(sid * rpt, rpt)])

    return deg(dstw, zeros, ones)


def _dinv_block(degp):
    deg = degp[0, :, 0:1] + degp[1, :, 0:1] + 1.0
    return lax.rsqrt(deg)


def _row_block(n):
    for r in (2000, 1600, 1250, 1000, 800, 640, 625, 500, 400, 250, 200, 125, 100):
        if n % r == 0:
            return r
    return n


def _tc_layer1(x, W1, degp, n_pad):
    N, D_IN = x.shape
    D_HID = W1.shape[1]
    R = _row_block(N)

    def body(x_ref, w1_ref, degp_ref, hs_ref):
        dinv = _dinv_block(degp_ref[...])
        h = jnp.dot(x_ref[...], w1_ref[...], preferred_element_type=jnp.float32)
        hs_ref[...] = (h * dinv).astype(jnp.bfloat16)

    return pl.pallas_call(
        body,
        grid=(N // R,),
        in_specs=[
            pl.BlockSpec((R, D_IN), lambda j: (j, 0)),
            pl.BlockSpec((D_IN, D_HID), lambda j: (0, 0)),
            pl.BlockSpec((NC, R, 8), lambda j: (0, j, 0)),
        ],
        out_specs=pl.BlockSpec((R, D_HID), lambda j: (j, 0)),
        out_shape=jax.ShapeDtypeStruct((N, D_HID), jnp.bfloat16),
    )(x, W1, degp)


def _tc_layer2(hs, aggp, degp, b1, W2, n_pad):
    N, D_HID = hs.shape
    D_OUT = W2.shape[1]
    R = _row_block(N)

    def body(hs_ref, aggp_ref, degp_ref, b1_ref, w2_ref, ts_ref):
        dinv = _dinv_block(degp_ref[...])
        s = (aggp_ref[0] + aggp_ref[1] + hs_ref[...]).astype(jnp.float32)
        h1 = jnp.maximum(s * dinv + b1_ref[...], 0.0)
        t = jnp.dot(h1, w2_ref[...], preferred_element_type=jnp.float32)
        ts_ref[...] = (t * dinv).astype(jnp.bfloat16)

    return pl.pallas_call(
        body,
        grid=(N // R,),
        in_specs=[
            pl.BlockSpec((R, D_HID), lambda j: (j, 0)),
            pl.BlockSpec((NC, R, D_HID), lambda j: (0, j, 0)),
            pl.BlockSpec((NC, R, 8), lambda j: (0, j, 0)),
            pl.BlockSpec((1, D_HID), lambda j: (0, 0)),
            pl.BlockSpec((D_HID, D_OUT), lambda j: (0, 0)),
        ],
        out_specs=pl.BlockSpec((R, D_OUT), lambda j: (j, 0)),
        out_shape=jax.ShapeDtypeStruct((N, D_OUT), jnp.bfloat16),
    )(hs, aggp, degp, b1.reshape(1, D_HID), W2)


def _tc_head(ts, aggp, degp, b2, Wfc, bfc, n_pad):
    N, D_OUT = ts.shape
    R = _row_block(N)
    G = N // R

    def body(ts_ref, aggp_ref, degp_ref, b2_ref, wfc_ref, bfc_ref, out_ref, acc_ref):
        j = pl.program_id(0)
        dinv = _dinv_block(degp_ref[...])
        s = (aggp_ref[0] + aggp_ref[1] + ts_ref[...]).astype(jnp.float32)
        h2 = jnp.maximum(s * dinv + b2_ref[...], 0.0)
        csum = jnp.sum(h2, axis=0, keepdims=True)

        @pl.when(j == 0)
        def _():
            acc_ref[...] = csum

        @pl.when(j > 0)
        def _():
            acc_ref[...] += csum

        @pl.when(j == G - 1)
        def _():
            g = acc_ref[...] * (1.0 / N)
            z = jnp.dot(g, wfc_ref[...], preferred_element_type=jnp.float32)
            z = z + bfc_ref[...]
            out_ref[...] = 1.0 / (1.0 + jnp.exp(-z))

    return pl.pallas_call(
        body,
        grid=(G,),
        in_specs=[
            pl.BlockSpec((R, D_OUT), lambda j: (j, 0)),
            pl.BlockSpec((NC, R, D_OUT), lambda j: (0, j, 0)),
            pl.BlockSpec((NC, R, 8), lambda j: (0, j, 0)),
            pl.BlockSpec((1, D_OUT), lambda j: (0, 0)),
            pl.BlockSpec((D_OUT, 1), lambda j: (0, 0)),
            pl.BlockSpec((1, 1), lambda j: (0, 0)),
        ],
        out_specs=pl.BlockSpec((1, 1), lambda j: (0, 0)),
        out_shape=jax.ShapeDtypeStruct((1, 1), jnp.float32),
        scratch_shapes=[pltpu.VMEM((1, D_OUT), jnp.float32)],
    )(ts, aggp, degp, b2.reshape(1, D_OUT), Wfc, bfc.reshape(1, 1))


def kernel(x, edge_index, W1, b1, W2, b2, Wfc, bfc):
    N = x.shape[0]
    E = edge_index.shape[1]
    NW = NC * NS
    L = -(-E // (NW * C_DEG)) * C_DEG
    e_pad = NW * L
    n_pad = -(-(N + 1) // 128) * 128

    src = edge_index[0]
    dst = edge_index[1]
    src_f = jnp.concatenate(
        [src, jnp.zeros((e_pad - E,), jnp.int32)]).reshape(NW, L)
    dst_f = jnp.concatenate(
        [dst, jnp.full((e_pad - E,), N, jnp.int32)]).reshape(NW, L)
    src_p = src_f.reshape(NW, L // C_AGG, C_AGG)
    dst_p = dst_f.reshape(NW, L // C_AGG, C_AGG)
    dst_d = dst_f.reshape(NW, L // C_DEG, C_DEG)

    def pad_rows(a):
        nt = -(-a.shape[0] // NS) * NS
        if nt == a.shape[0]:
            return a
        return jnp.concatenate(
            [a, jnp.zeros((nt - a.shape[0], a.shape[1]), a.dtype)])

    degp = _sc_degree(dst_d, n_pad)
    hs = _tc_layer1(x, W1, degp, n_pad)
    agg1 = _sc_aggregate(pad_rows(hs), src_p, dst_p, n_pad)
    ts = _tc_layer2(hs, agg1, degp, b1, W2, n_pad)
    agg2 = _sc_aggregate(pad_rows(ts), src_p, dst_p, n_pad)
    out = _tc_head(ts, agg2, degp, b2, Wfc, bfc, n_pad)
    return out.reshape(1)
